# baseline (speedup 1.0000x reference)
import math

import numpy as np
import jax
import jax.numpy as jnp
from jax import lax
from jax.experimental import pallas as pl
from jax.experimental.pallas import tpu as pltpu
from jax.experimental.pallas import tpu_sc as plsc

VOCAB = 1000000
EMB = 64
MAX_LEN = 512
BATCH = 4096
SEQ = 200
SCALE = math.sqrt(EMB)

NC = 2
NS = 16
L = 16
NW = NC * NS
ROWS = BATCH * SEQ
RPW = ROWS // NW
CG = 128
NCH = RPW // CG
NB = 8
AH = 4


def _pos_embedding_np():
    rng = np.exp(-np.arange(0, EMB, 2, dtype=np.float64) * math.log(10000) / EMB)
    pos = np.arange(0, MAX_LEN, dtype=np.float64).reshape(MAX_LEN, 1)
    pe = np.zeros((MAX_LEN, EMB), dtype=np.float32)
    pe[:, 0::2] = np.sin(pos * rng).astype(np.float32)
    pe[:, 1::2] = np.cos(pos * rng).astype(np.float32)
    return pe[:SEQ]


_PE = _pos_embedding_np()


def _sc_body(tokens_hbm, pe_hbm, table_hbm, out_hbm, pe_v, tok_v, row_v,
             gsem, ssem):
    wid = lax.axis_index("s") * NC + lax.axis_index("c")
    r0 = pl.multiple_of(wid * RPW, 8)

    pltpu.sync_copy(pe_hbm, pe_v)
    pltpu.sync_copy(tokens_hbm.at[pl.ds(r0, RPW)], tok_v)

    def start_gather(c, s):
        pltpu.async_copy(table_hbm.at[tok_v.at[pl.ds(CG * c, CG)]],
                         row_v.at[s], gsem.at[s])

    def wait_gather(c, s):
        pltpu.make_async_copy(table_hbm.at[tok_v.at[pl.ds(CG * c, CG)]],
                              row_v.at[s], gsem.at[s]).wait()

    def out_dst(c):
        return out_hbm.at[pl.ds(r0 + CG * c, CG)]

    def start_store(c, s):
        pltpu.async_copy(row_v.at[s], out_dst(c), ssem.at[s])

    def wait_store(c, s):
        pltpu.make_async_copy(row_v.at[s], out_dst(c), ssem.at[s]).wait()

    for pp in range(AH):
        start_gather(pp, pp)

    def c_body(c, carry):
        s = lax.rem(c, NB)
        s2 = lax.rem(c + AH, NB)

        @pl.when(c + AH < NCH)
        def _():
            @pl.when(c >= NB - AH)
            def _():
                wait_store(c + AH - NB, s2)

            start_gather(c + AH, s2)

        wait_gather(c, s)

        p0 = lax.rem(r0 + CG * c, SEQ)

        def r_body(r, p):
            for qq in range(EMB // L):
                row_v[s, r, pl.ds(qq * L, L)] = (
                    row_v[s, r, pl.ds(qq * L, L)] * SCALE
                    + pe_v[p, pl.ds(qq * L, L)])
            p = p + 1
            return lax.select(p >= SEQ, p - SEQ, p)

        lax.fori_loop(0, CG, r_body, p0, unroll=8)
        start_store(c, s)
        return carry

    lax.fori_loop(0, NCH, c_body, 0)

    for k in range(NB):
        c = NCH - NB + k
        wait_store(c, lax.rem(jnp.int32(c), NB))


def kernel(tokens, embedding_weight):
    tokens_flat = tokens.astype(jnp.int32).reshape(ROWS)
    pe = jnp.asarray(_PE)
    mesh = plsc.VectorSubcoreMesh(
        core_axis_name="c", subcore_axis_name="s", num_cores=NC,
        num_subcores=NS)
    k = pl.kernel(
        _sc_body,
        out_type=jax.ShapeDtypeStruct((ROWS, EMB), jnp.float32),
        mesh=mesh,
        scratch_types=[
            pltpu.VMEM((SEQ, EMB), jnp.float32),
            pltpu.VMEM((RPW,), jnp.int32),
            pltpu.VMEM((NB, CG, EMB), jnp.float32),
            pltpu.SemaphoreType.DMA((NB,)),
            pltpu.SemaphoreType.DMA((NB,)),
        ],
        compiler_params=pltpu.CompilerParams(use_tc_tiling_on_sc=False),
    )
    out = k(tokens_flat, pe, embedding_weight)
    return out.reshape(BATCH, SEQ, EMB)


# speedup vs baseline: 1.0450x; 1.0450x over previous
import math

import numpy as np
import jax
import jax.numpy as jnp
from jax import lax
from jax.experimental import pallas as pl
from jax.experimental.pallas import tpu as pltpu
from jax.experimental.pallas import tpu_sc as plsc

VOCAB = 1000000
EMB = 64
MAX_LEN = 512
BATCH = 4096
SEQ = 200
SCALE = math.sqrt(EMB)

NC = 2
NS = 16
L = 16
NW = NC * NS
ROWS = BATCH * SEQ
RPW = ROWS // NW
CG = 128
NCH = RPW // CG
NB = 3
AH = 2
NO = 2


def _pos_embedding_np():
    rng = np.exp(-np.arange(0, EMB, 2, dtype=np.float64) * math.log(10000) / EMB)
    pos = np.arange(0, MAX_LEN, dtype=np.float64).reshape(MAX_LEN, 1)
    pe = np.zeros((MAX_LEN, EMB), dtype=np.float32)
    pe[:, 0::2] = np.sin(pos * rng).astype(np.float32)
    pe[:, 1::2] = np.cos(pos * rng).astype(np.float32)
    return pe[:SEQ]


_PE = _pos_embedding_np()

_DNUMS = lax.GatherDimensionNumbers(
    offset_dims=(), collapsed_slice_dims=(0,), start_index_map=(0,))
_PIB = lax.GatherScatterMode.PROMISE_IN_BOUNDS


def _bcast_lane(vec, k):
    idx = (jnp.full((L,), 0, jnp.int32) + k).reshape(L, 1)
    return lax.gather(vec, idx, _DNUMS, (1,), mode=_PIB)


def _sc_body(tokens_hbm, pe_hbm, wtab, out_hbm, pe_v, tok_v, widx, row_v,
             obuf, gsem, ssem):
    wid = lax.axis_index("s") * NC + lax.axis_index("c")
    r0 = pl.multiple_of(wid * RPW, 1024)

    pltpu.sync_copy(pe_hbm, pe_v)
    pltpu.sync_copy(tokens_hbm.at[pl.ds(r0, RPW)], tok_v)

    def start_gather(c, s):
        for k in range(CG // L):
            t = tok_v[pl.ds(CG * c + 16 * k, L)]
            widx[s, pl.ds(16 * k, L)] = lax.shift_right_logical(t, 1)
        pltpu.async_copy(wtab.at[widx.at[s]], row_v.at[s], gsem.at[s])

    def wait_gather(c, s):
        pltpu.make_async_copy(wtab.at[widx.at[s]], row_v.at[s],
                              gsem.at[s]).wait()

    def out_dst(c):
        return out_hbm.at[pl.ds(r0 + CG * c, CG)]

    def start_store(c, ob):
        pltpu.async_copy(obuf.at[ob], out_dst(c), ssem.at[ob])

    def wait_store(c, ob):
        pltpu.make_async_copy(obuf.at[ob], out_dst(c), ssem.at[ob]).wait()

    for pp in range(AH):
        start_gather(pp, pp)

    def c_body(c, carry):
        s = lax.rem(c, NB)
        s2 = lax.rem(c + AH, NB)
        ob = lax.rem(c, NO)

        @pl.when(c + AH < NCH)
        def _():
            start_gather(c + AH, s2)

        wait_gather(c, s)

        @pl.when(c >= NO)
        def _():
            wait_store(c - NO, ob)

        p0 = lax.rem(r0 + CG * c, SEQ)

        def grp_body(gk, p):
            sel16 = lax.bitwise_and(tok_v[pl.ds(CG * c + 16 * gk, L)],
                                    jnp.int32(1))
            for k in range(L):
                r = 16 * gk + k
                m = _bcast_lane(sel16, k) == 1
                for qq in range(EMB // L):
                    lo = row_v[s, r, pl.ds(qq * L, L)]
                    hi = row_v[s, r, pl.ds(EMB + qq * L, L)]
                    obuf[ob, r, pl.ds(qq * L, L)] = (
                        jnp.where(m, hi, lo) * SCALE
                        + pe_v[pl.ds(p * EMB + qq * L, L)])
                p = p + 1
                p = lax.select(p >= SEQ, p - SEQ, p)
            return p

        lax.fori_loop(0, CG // L, grp_body, p0)
        start_store(c, ob)
        return carry

    lax.fori_loop(0, NCH, c_body, 0)

    for k in range(NO):
        c = NCH - NO + k
        wait_store(c, lax.rem(jnp.int32(c), NO))


def kernel(tokens, embedding_weight):
    tokens_flat = tokens.astype(jnp.int32).reshape(ROWS)
    wtab = embedding_weight.reshape(VOCAB // 2, 2 * EMB)
    pe = jnp.asarray(_PE).reshape(-1)
    mesh = plsc.VectorSubcoreMesh(
        core_axis_name="c", subcore_axis_name="s", num_cores=NC,
        num_subcores=NS)
    k = pl.kernel(
        _sc_body,
        out_type=jax.ShapeDtypeStruct((ROWS, EMB), jnp.float32),
        mesh=mesh,
        scratch_types=[
            pltpu.VMEM((SEQ * EMB,), jnp.float32),
            pltpu.VMEM((RPW,), jnp.int32),
            pltpu.VMEM((NB, CG), jnp.int32),
            pltpu.VMEM((NB, CG, 2 * EMB), jnp.float32),
            pltpu.VMEM((NO, CG, EMB), jnp.float32),
            pltpu.SemaphoreType.DMA((NB,)),
            pltpu.SemaphoreType.DMA((NO,)),
        ],
        compiler_params=pltpu.CompilerParams(needs_layout_passes=False),
    )
    out = k(tokens_flat, pe, wtab)
    return out.reshape(BATCH, SEQ, EMB)
